# baseline (speedup 1.0000x reference)
import functools

import jax
import jax.numpy as jnp
from jax import lax
from jax.experimental import pallas as pl
from jax.experimental.pallas import tpu as pltpu
from jax.experimental.pallas import tpu_sc as plsc

B = 4096
L = 32
V = 185
PAD = 184
S1 = 256
S2 = 64
KW = 192
NH = 4
HW = NH * KW
BB = 2048
NW = 32
SPT = B // NW
RND = 32


def _sc_hist_body(idx_hbm, col_hbm, sob_hbm, h_hbm, idx_v, col_v, sob_v,
                  h_a, h_b, sem):
    cc = lax.axis_index("c")
    ss = lax.axis_index("s")
    wid = ss * 2 + cc
    base = wid * SPT
    in1 = pltpu.async_copy(idx_hbm.at[wid], idx_v, sem)
    in2 = pltpu.async_copy(col_hbm.at[wid], col_v, sem)
    in3 = pltpu.async_copy(sob_hbm.at[wid], sob_v, sem)

    z16 = jnp.zeros((16,), jnp.float32)
    iota16 = lax.iota(jnp.int32, 16)
    ones16 = jnp.ones((16,), jnp.float32)

    def zero_buf(hv):
        def zero_body(b, carry):
            for j in range(HW // 16):
                hv[b, pl.ds(j * 16, 16)] = z16
            return carry
        lax.fori_loop(0, RND, zero_body, 0)

    def scatter_round(hv, r):
        def scat_body(l, carry):
            l16 = jnp.full((16,), l, jnp.int32)
            for chunk in range(RND // 16):
                off = r * RND + chunk * 16
                s16 = off + iota16
                vi = plsc.load_gather(idx_v, [s16, l16])
                cv = plsc.load_gather(col_v, [s16, l16])
                sv = plsc.load_gather(sob_v, [s16, l16])
                b16 = chunk * 16 + iota16
                plsc.addupdate_scatter(hv, [b16, vi], ones16)
                plsc.addupdate_scatter(hv, [b16, vi + KW], cv)
                plsc.addupdate_scatter(hv, [b16, vi + 2 * KW], sv)
                plsc.addupdate_scatter(hv, [b16, vi + 3 * KW], cv * sv)
            return carry
        lax.fori_loop(0, L, scat_body, 0)

    bufs = [h_a, h_b]
    pending = [None, None]
    for r in range(SPT // RND):
        hv = bufs[r % 2]
        if pending[r % 2] is not None:
            pending[r % 2].wait()
        zero_buf(hv)
        if r == 0:
            in1.wait()
            in2.wait()
            in3.wait()
        scatter_round(hv, r)
        pending[r % 2] = pltpu.async_copy(
            hv, h_hbm.at[pl.ds(base + r * RND, RND)], sem)
    for cp in pending:
        cp.wait()


def _tc_mm_body(h_ref, wtm_ref, t1_ref, t2_ref, t3_ref, t4_ref, wt_ref,
                o1_ref, o2_ref, o3_ref, o4_ref):
    h = h_ref[...]
    dn = (((0,), (1,)), ((), ()))
    o1_ref[...] = jnp.dot(h[:, 0:KW], t1_ref[...],
                          preferred_element_type=jnp.float32)
    o2_ref[...] = (jnp.dot(h[:, KW:2 * KW], t2_ref[...],
                           preferred_element_type=jnp.float32)
                   + wtm_ref[...] * wt_ref[...])
    o3_ref[...] = lax.dot_general(t3_ref[...], h[:, 2 * KW:3 * KW], dn,
                                  preferred_element_type=jnp.float32)
    o4_ref[...] = lax.dot_general(t4_ref[...], h[:, 3 * KW:4 * KW], dn,
                                  preferred_element_type=jnp.float32)


@jax.jit
def kernel(pst_idx, color_sign, sob_sign, wtm, T_fs, T_va, T_ha, T_ra,
           W_tempo):
    idx3 = pst_idx.reshape(NW, SPT, L)
    col3 = color_sign.reshape(NW, SPT, L)
    sob3 = sob_sign.reshape(NW, SPT, L)

    mesh = plsc.VectorSubcoreMesh(core_axis_name="c", subcore_axis_name="s")
    hist = pl.kernel(
        _sc_hist_body,
        out_type=jax.ShapeDtypeStruct((B, HW), jnp.float32),
        mesh=mesh,
        compiler_params=pltpu.CompilerParams(needs_layout_passes=False),
        scratch_types=[
            pltpu.VMEM((SPT, L), jnp.int32),
            pltpu.VMEM((SPT, L), jnp.float32),
            pltpu.VMEM((SPT, L), jnp.float32),
            pltpu.VMEM((RND, HW), jnp.float32),
            pltpu.VMEM((RND, HW), jnp.float32),
            pltpu.SemaphoreType.DMA,
        ],
    )(idx3, col3, sob3)

    t1 = jnp.zeros((KW, S1), jnp.float32).at[:V].set(T_fs)
    t2 = jnp.zeros((KW, S1), jnp.float32).at[:V].set(T_va)
    t3 = jnp.zeros((KW, S2), jnp.float32).at[:V].set(T_ha)
    t4 = jnp.zeros((KW, S2), jnp.float32).at[:V].set(T_ra)
    wt = W_tempo.reshape(1, S1)

    tspec = lambda d: pl.BlockSpec((KW, d), lambda i: (0, 0))
    out = pl.pallas_call(
        _tc_mm_body,
        grid=(B // BB,),
        in_specs=[
            pl.BlockSpec((BB, HW), lambda i: (i, 0)),
            pl.BlockSpec((BB, 1), lambda i: (i, 0)),
            tspec(S1), tspec(S1), tspec(S2), tspec(S2),
            pl.BlockSpec((1, S1), lambda i: (0, 0)),
        ],
        out_specs=[
            pl.BlockSpec((BB, S1), lambda i: (i, 0)),
            pl.BlockSpec((BB, S1), lambda i: (i, 0)),
            pl.BlockSpec((S2, BB), lambda i: (0, i)),
            pl.BlockSpec((S2, BB), lambda i: (0, i)),
        ],
        out_shape=[
            jax.ShapeDtypeStruct((B, S1), jnp.float32),
            jax.ShapeDtypeStruct((B, S1), jnp.float32),
            jax.ShapeDtypeStruct((S2, B), jnp.float32),
            jax.ShapeDtypeStruct((S2, B), jnp.float32),
        ],
    )(hist, wtm, t1, t2, t3, t4, wt)
    return (out[0], out[1], out[2].T, out[3].T)


# speedup vs baseline: 1.2425x; 1.2425x over previous
import functools

import jax
import jax.numpy as jnp
from jax import lax
from jax.experimental import pallas as pl
from jax.experimental.pallas import tpu as pltpu
from jax.experimental.pallas import tpu_sc as plsc

B = 4096
L = 32
V = 185
PAD = 184
S1 = 256
S2 = 64
KW = 192
NH = 4
HW = NH * KW
BB = 2048
NW = 32
SPT = B // NW
RND = 32


def _sc_hist_body(idx_hbm, col_hbm, sob_hbm, h_hbm, idx_v, col_v, sob_v,
                  h_a, h_b, sem):
    cc = lax.axis_index("c")
    ss = lax.axis_index("s")
    wid = ss * 2 + cc
    base = wid * SPT
    in1 = pltpu.async_copy(idx_hbm.at[wid], idx_v, sem)
    in2 = pltpu.async_copy(col_hbm.at[wid], col_v, sem)
    in3 = pltpu.async_copy(sob_hbm.at[wid], sob_v, sem)

    z16 = jnp.zeros((16,), jnp.float32)
    iota16 = lax.iota(jnp.int32, 16)
    ones16 = jnp.ones((16,), jnp.float32)

    def zero_buf(hv):
        def zero_body(b, carry):
            for j in range(HW // 16):
                hv[b, pl.ds(j * 16, 16)] = z16
            return carry
        lax.fori_loop(0, RND, zero_body, 0)

    def scatter_round(hv, r):
        def scat_body(l, carry):
            for chunk in range(RND // 16):
                off = r * RND + chunk * 16
                vi = idx_v[l, pl.ds(off, 16)]
                cv = col_v[l, pl.ds(off, 16)]
                sv = sob_v[l, pl.ds(off, 16)]
                b16 = chunk * 16 + iota16
                plsc.addupdate_scatter(hv, [b16, vi], ones16)
                plsc.addupdate_scatter(hv, [b16, vi + KW], cv)
                plsc.addupdate_scatter(hv, [b16, vi + 2 * KW], sv)
                plsc.addupdate_scatter(hv, [b16, vi + 3 * KW], cv * sv)
            return carry
        lax.fori_loop(0, L, scat_body, 0)

    bufs = [h_a, h_b]
    pending = [None, None]
    for r in range(SPT // RND):
        hv = bufs[r % 2]
        if pending[r % 2] is not None:
            pending[r % 2].wait()
        zero_buf(hv)
        if r == 0:
            in1.wait()
            in2.wait()
            in3.wait()
        scatter_round(hv, r)
        pending[r % 2] = pltpu.async_copy(
            hv, h_hbm.at[pl.ds(base + r * RND, RND)], sem)
    for cp in pending:
        cp.wait()


def _tc_mm_body(h_ref, wtm_ref, t1_ref, t2_ref, t3_ref, t4_ref, wt_ref,
                o1_ref, o2_ref, o3_ref, o4_ref):
    h = h_ref[...]
    dn = (((0,), (1,)), ((), ()))
    o1_ref[...] = jnp.dot(h[:, 0:KW], t1_ref[...],
                          preferred_element_type=jnp.float32)
    o2_ref[...] = (jnp.dot(h[:, KW:2 * KW], t2_ref[...],
                           preferred_element_type=jnp.float32)
                   + wtm_ref[...] * wt_ref[...])
    o3_ref[...] = lax.dot_general(t3_ref[...], h[:, 2 * KW:3 * KW], dn,
                                  preferred_element_type=jnp.float32)
    o4_ref[...] = lax.dot_general(t4_ref[...], h[:, 3 * KW:4 * KW], dn,
                                  preferred_element_type=jnp.float32)


@jax.jit
def kernel(pst_idx, color_sign, sob_sign, wtm, T_fs, T_va, T_ha, T_ra,
           W_tempo):
    idx3 = pst_idx.reshape(NW, SPT, L).transpose(0, 2, 1)
    col3 = color_sign.reshape(NW, SPT, L).transpose(0, 2, 1)
    sob3 = sob_sign.reshape(NW, SPT, L).transpose(0, 2, 1)

    mesh = plsc.VectorSubcoreMesh(core_axis_name="c", subcore_axis_name="s")
    hist = pl.kernel(
        _sc_hist_body,
        out_type=jax.ShapeDtypeStruct((B, HW), jnp.float32),
        mesh=mesh,
        compiler_params=pltpu.CompilerParams(needs_layout_passes=False),
        scratch_types=[
            pltpu.VMEM((L, SPT), jnp.int32),
            pltpu.VMEM((L, SPT), jnp.float32),
            pltpu.VMEM((L, SPT), jnp.float32),
            pltpu.VMEM((RND, HW), jnp.float32),
            pltpu.VMEM((RND, HW), jnp.float32),
            pltpu.SemaphoreType.DMA,
        ],
    )(idx3, col3, sob3)

    t1 = jnp.zeros((KW, S1), jnp.float32).at[:V].set(T_fs)
    t2 = jnp.zeros((KW, S1), jnp.float32).at[:V].set(T_va)
    t3 = jnp.zeros((KW, S2), jnp.float32).at[:V].set(T_ha)
    t4 = jnp.zeros((KW, S2), jnp.float32).at[:V].set(T_ra)
    wt = W_tempo.reshape(1, S1)

    tspec = lambda d: pl.BlockSpec((KW, d), lambda i: (0, 0))
    out = pl.pallas_call(
        _tc_mm_body,
        grid=(B // BB,),
        in_specs=[
            pl.BlockSpec((BB, HW), lambda i: (i, 0)),
            pl.BlockSpec((BB, 1), lambda i: (i, 0)),
            tspec(S1), tspec(S1), tspec(S2), tspec(S2),
            pl.BlockSpec((1, S1), lambda i: (0, 0)),
        ],
        out_specs=[
            pl.BlockSpec((BB, S1), lambda i: (i, 0)),
            pl.BlockSpec((BB, S1), lambda i: (i, 0)),
            pl.BlockSpec((S2, BB), lambda i: (0, i)),
            pl.BlockSpec((S2, BB), lambda i: (0, i)),
        ],
        out_shape=[
            jax.ShapeDtypeStruct((B, S1), jnp.float32),
            jax.ShapeDtypeStruct((B, S1), jnp.float32),
            jax.ShapeDtypeStruct((S2, B), jnp.float32),
            jax.ShapeDtypeStruct((S2, B), jnp.float32),
        ],
    )(hist, wtm, t1, t2, t3, t4, wt)
    return (out[0], out[1], out[2].T, out[3].T)
